# baseline (speedup 1.0000x reference)
import functools

import jax
import jax.numpy as jnp
from jax import lax
from jax.experimental import pallas as pl
from jax.experimental.pallas import tpu as pltpu
from jax.experimental.pallas import tpu_sc as plsc

N = 10000
NP = 10240
D = 128
E = 320000
NW = 32
CHUNK = 50
CH = 200
EPW = CH * CHUNK
CH0 = 200
CH1 = 200
PHROWS = 40
NBUF = 5
STRIPE = NP // 16

_mesh = plsc.VectorSubcoreMesh(core_axis_name="c", subcore_axis_name="s")


@functools.partial(
    pl.kernel,
    mesh=_mesh,
    out_type=jax.ShapeDtypeStruct((2, NP, D), jnp.float32),
    scratch_types=[
        pltpu.VMEM((PHROWS, CHUNK), jnp.int32),
        pltpu.VMEM((PHROWS, CHUNK), jnp.int32),
        pltpu.VMEM((NBUF, CHUNK, D), jnp.float32),
        pltpu.VMEM_SHARED((NP, D), jnp.float32),
        pltpu.SemaphoreType.DMA,
        pltpu.SemaphoreType.DMA,
        pltpu.SemaphoreType.DMA,
        pltpu.SemaphoreType.DMA,
        pltpu.SemaphoreType.DMA,
        pltpu.SemaphoreType.DMA,
        pltpu.SemaphoreType.DMA,
        pltpu.SemaphoreType.DMA,
        pltpu.SemaphoreType.DMA,
        pltpu.SemaphoreType.DMA,
    ],
)
def _sc_segsum(h_hbm, src_hbm, dst_hbm, zeros_hbm, out_hbm,
               src_v, dst_v, rows_v, acc_sh, *sems):
    gsem = sems[:NBUF]
    ssem = sems[NBUF:]
    cid = lax.axis_index("c")
    sid = lax.axis_index("s")
    pltpu.sync_copy(zeros_hbm, acc_sh.at[pl.ds(sid * STRIPE, STRIPE)])
    plsc.subcore_barrier()

    def run_edges(base_row, ch):
        assert ch % PHROWS == 0 and PHROWS % NBUF == 0
        for p in range(ch // PHROWS):
            base = base_row + p * PHROWS
            pltpu.sync_copy(src_hbm.at[pl.ds(base, PHROWS)], src_v)
            pltpu.sync_copy(dst_hbm.at[pl.ds(base, PHROWS)], dst_v)
            for b in range(NBUF):
                pltpu.async_copy(h_hbm.at[src_v.at[b]], rows_v.at[b],
                                 gsem[b])

            def body(i, carry):
                j0 = i * NBUF
                for b in range(NBUF):
                    pltpu.make_async_copy(
                        h_hbm.at[src_v.at[j0 + b]], rows_v.at[b],
                        gsem[b]).wait()
                    pltpu.async_copy(rows_v.at[b],
                                     acc_sh.at[dst_v.at[j0 + b]],
                                     ssem[b], add=True)
                for b in range(NBUF):
                    pltpu.make_async_copy(
                        rows_v.at[b], acc_sh.at[dst_v.at[j0 + b]],
                        ssem[b]).wait()

                    @pl.when(i < PHROWS // NBUF - 1)
                    def _():
                        pltpu.async_copy(
                            h_hbm.at[src_v.at[j0 + NBUF + b]],
                            rows_v.at[b], gsem[b])
                return carry

            lax.fori_loop(0, PHROWS // NBUF, body, 0)

    if CH0 > 0:
        @pl.when(cid == 0)
        def _():
            run_edges(sid * CH0, CH0)

    if CH1 > 0:
        @pl.when(cid == 1)
        def _():
            run_edges(16 * CH0 + sid * CH1, CH1)

    plsc.subcore_barrier()
    pltpu.sync_copy(acc_sh.at[pl.ds(sid * STRIPE, STRIPE)],
                    out_hbm.at[cid, pl.ds(sid * STRIPE, STRIPE)])


@functools.partial(
    pl.kernel,
    mesh=_mesh,
    out_type=jax.ShapeDtypeStruct((2, NP), jnp.float32),
    scratch_types=[
        pltpu.VMEM((CH, CHUNK), jnp.int32),
        pltpu.VMEM((CHUNK,), jnp.float32),
        pltpu.VMEM_SHARED((NP,), jnp.float32),
    ],
)
def _sc_degree(dst_hbm, ones_hbm, zeros1_hbm, out_hbm, dst_v, ones_v, deg_sh):
    cid = lax.axis_index("c")
    sid = lax.axis_index("s")
    wid = sid * 2 + cid
    pltpu.sync_copy(zeros1_hbm, deg_sh.at[pl.ds(sid * STRIPE, STRIPE)])
    pltpu.sync_copy(ones_hbm, ones_v)
    pltpu.sync_copy(dst_hbm.at[pl.ds(wid * CH, CH)], dst_v)
    plsc.subcore_barrier()

    def body(j, carry):
        pltpu.sync_copy(ones_v, deg_sh.at[dst_v.at[j]], add=True)
        return carry

    lax.fori_loop(0, CH, body, 0)
    plsc.subcore_barrier()

    @pl.when(sid == 0)
    def _():
        pltpu.sync_copy(deg_sh, out_hbm.at[cid])


NBLK = 1000
NGRID = N // NBLK


def _mm_scale_body(x_ref, w_ref, deg_ref, o_ref, dinv_ref):
    d = deg_ref[0] + deg_ref[1] + 1.0
    dinv = lax.rsqrt(d)
    dinv_ref[...] = dinv
    h = jnp.dot(x_ref[...], w_ref[...], preferred_element_type=jnp.float32)
    o_ref[...] = h * dinv


def _tc_mm_scale(x, W, deg2):
    return pl.pallas_call(
        _mm_scale_body,
        grid=(NGRID,),
        in_specs=[
            pl.BlockSpec((NBLK, D), lambda i: (i, 0)),
            pl.BlockSpec((D, D), lambda i: (0, 0)),
            pl.BlockSpec((2, NBLK, 1), lambda i: (0, i, 0)),
        ],
        out_specs=[
            pl.BlockSpec((NBLK, D), lambda i: (i, 0)),
            pl.BlockSpec((NBLK, 1), lambda i: (i, 0)),
        ],
        out_shape=[
            jax.ShapeDtypeStruct((NP, D), jnp.float32),
            jax.ShapeDtypeStruct((NP, 1), jnp.float32),
        ],
    )(x, W, deg2)


def _combine_mm_body(acc_ref, hp_ref, dinv_ref, b_ref, w_ref, o_ref):
    dinv = dinv_ref[...]
    enc = dinv * (acc_ref[0] + acc_ref[1] + hp_ref[...]) + b_ref[...]
    o_ref[...] = jnp.dot(enc, w_ref[...],
                         preferred_element_type=jnp.float32) * dinv


def _tc_combine_mm(acc2, hp, dinv, b, W):
    return pl.pallas_call(
        _combine_mm_body,
        grid=(NGRID,),
        in_specs=[
            pl.BlockSpec((2, NBLK, D), lambda i: (0, i, 0)),
            pl.BlockSpec((NBLK, D), lambda i: (i, 0)),
            pl.BlockSpec((NBLK, 1), lambda i: (i, 0)),
            pl.BlockSpec((1, D), lambda i: (0, 0)),
            pl.BlockSpec((D, D), lambda i: (0, 0)),
        ],
        out_specs=pl.BlockSpec((NBLK, D), lambda i: (i, 0)),
        out_shape=jax.ShapeDtypeStruct((NP, D), jnp.float32),
    )(acc2, hp, dinv, b, W)


def _final_body(acc_ref, hp_ref, dinv_ref, b_ref, o_ref):
    o_ref[...] = (dinv_ref[...] * (acc_ref[0] + acc_ref[1] + hp_ref[...])
                  + b_ref[...])


def _tc_final(acc2, hp, dinv, b):
    return pl.pallas_call(
        _final_body,
        grid=(NGRID,),
        in_specs=[
            pl.BlockSpec((2, NBLK, D), lambda i: (0, i, 0)),
            pl.BlockSpec((NBLK, D), lambda i: (i, 0)),
            pl.BlockSpec((NBLK, 1), lambda i: (i, 0)),
            pl.BlockSpec((1, D), lambda i: (0, 0)),
        ],
        out_specs=pl.BlockSpec((NBLK, D), lambda i: (i, 0)),
        out_shape=jax.ShapeDtypeStruct((N, D), jnp.float32),
    )(acc2, hp, dinv, b)


def kernel(x, edge_index, W_enc, b_enc, W_dec, b_dec):
    src_p = edge_index[0].astype(jnp.int32).reshape(NW * CH, CHUNK)
    dst_p = edge_index[1].astype(jnp.int32).reshape(NW * CH, CHUNK)
    zeros = jnp.zeros((STRIPE, D), jnp.float32)
    zeros1 = jnp.zeros((STRIPE,), jnp.float32)
    ones1 = jnp.ones((CHUNK,), jnp.float32)

    deg2 = _sc_degree(dst_p, ones1, zeros1)
    h1, dinv = _tc_mm_scale(x, W_enc,
                            deg2.reshape(2, NP, 1))
    acc1 = _sc_segsum(h1, src_p, dst_p, zeros)
    h2 = _tc_combine_mm(acc1, h1, dinv,
                        b_enc.reshape(1, D), W_dec)
    acc2 = _sc_segsum(h2, src_p, dst_p, zeros)
    return _tc_final(acc2, h2, dinv, b_dec.reshape(1, D))


# speedup vs baseline: 1.0330x; 1.0330x over previous
import functools

import jax
import jax.numpy as jnp
from jax import lax
from jax.experimental import pallas as pl
from jax.experimental.pallas import tpu as pltpu
from jax.experimental.pallas import tpu_sc as plsc

N = 10000
NP = 10240
D = 128
E = 320000
NW = 32
CHUNK = 50
CH = 200
EPW = CH * CHUNK
CH0 = 200
CH1 = 200
PHROWS = 40
NBUF = 5
STRIPE = NP // 16

_mesh = plsc.VectorSubcoreMesh(core_axis_name="c", subcore_axis_name="s")


@functools.partial(
    pl.kernel,
    mesh=_mesh,
    out_type=jax.ShapeDtypeStruct((2, NP, D), jnp.float32),
    scratch_types=[
        pltpu.VMEM((PHROWS, CHUNK), jnp.int32),
        pltpu.VMEM((PHROWS, CHUNK), jnp.int32),
        pltpu.VMEM((NBUF, CHUNK, D), jnp.float32),
        pltpu.VMEM_SHARED((NP, D), jnp.float32),
        pltpu.SemaphoreType.DMA,
        pltpu.SemaphoreType.DMA,
        pltpu.SemaphoreType.DMA,
        pltpu.SemaphoreType.DMA,
        pltpu.SemaphoreType.DMA,
        pltpu.SemaphoreType.DMA,
        pltpu.SemaphoreType.DMA,
        pltpu.SemaphoreType.DMA,
        pltpu.SemaphoreType.DMA,
        pltpu.SemaphoreType.DMA,
    ],
)
def _sc_segsum(h_hbm, src_hbm, dst_hbm, zeros_hbm, out_hbm,
               src_v, dst_v, rows_v, acc_sh, *sems):
    gsem = sems[:NBUF]
    ssem = sems[NBUF:]
    cid = lax.axis_index("c")
    sid = lax.axis_index("s")
    pltpu.sync_copy(zeros_hbm, acc_sh.at[pl.ds(sid * STRIPE, STRIPE)])
    plsc.subcore_barrier()

    def run_edges(base_row, ch):
        assert ch % PHROWS == 0 and PHROWS % NBUF == 0
        for p in range(ch // PHROWS):
            base = base_row + p * PHROWS
            pltpu.sync_copy(src_hbm.at[pl.ds(base, PHROWS)], src_v)
            pltpu.sync_copy(dst_hbm.at[pl.ds(base, PHROWS)], dst_v)
            for b in range(NBUF):
                pltpu.async_copy(h_hbm.at[src_v.at[b]], rows_v.at[b],
                                 gsem[b])

            def body(i, carry):
                j0 = i * NBUF
                for b in range(NBUF):
                    pltpu.make_async_copy(
                        h_hbm.at[src_v.at[j0 + b]], rows_v.at[b],
                        gsem[b]).wait()
                    pltpu.async_copy(rows_v.at[b],
                                     acc_sh.at[dst_v.at[j0 + b]],
                                     ssem[b], add=True)
                for b in range(NBUF):
                    pltpu.make_async_copy(
                        rows_v.at[b], acc_sh.at[dst_v.at[j0 + b]],
                        ssem[b]).wait()

                    @pl.when(i < PHROWS // NBUF - 1)
                    def _():
                        pltpu.async_copy(
                            h_hbm.at[src_v.at[j0 + NBUF + b]],
                            rows_v.at[b], gsem[b])
                return carry

            lax.fori_loop(0, PHROWS // NBUF, body, 0)

    if CH0 > 0:
        @pl.when(cid == 0)
        def _():
            run_edges(sid * CH0, CH0)

    if CH1 > 0:
        @pl.when(cid == 1)
        def _():
            run_edges(16 * CH0 + sid * CH1, CH1)

    plsc.subcore_barrier()
    pltpu.sync_copy(acc_sh.at[pl.ds(sid * STRIPE, STRIPE)],
                    out_hbm.at[cid, pl.ds(sid * STRIPE, STRIPE)])


@functools.partial(
    pl.kernel,
    mesh=_mesh,
    out_type=jax.ShapeDtypeStruct((2, NP), jnp.float32),
    scratch_types=[
        pltpu.VMEM((CH, CHUNK), jnp.int32),
        pltpu.VMEM((CHUNK,), jnp.float32),
        pltpu.VMEM_SHARED((NP,), jnp.float32),
        pltpu.SemaphoreType.DMA,
        pltpu.SemaphoreType.DMA,
        pltpu.SemaphoreType.DMA,
        pltpu.SemaphoreType.DMA,
    ],
)
def _sc_degree(dst_hbm, ones_hbm, zeros1_hbm, out_hbm, dst_v, ones_v,
               deg_sh, *dsems):
    cid = lax.axis_index("c")
    sid = lax.axis_index("s")
    wid = sid * 2 + cid
    pltpu.sync_copy(zeros1_hbm, deg_sh.at[pl.ds(sid * STRIPE, STRIPE)])
    pltpu.sync_copy(ones_hbm, ones_v)
    pltpu.sync_copy(dst_hbm.at[pl.ds(wid * CH, CH)], dst_v)
    plsc.subcore_barrier()

    NDB = len(dsems)
    for b in range(NDB):
        pltpu.async_copy(ones_v, deg_sh.at[dst_v.at[b]], dsems[b], add=True)

    def body(i, carry):
        j0 = i * NDB
        for b in range(NDB):
            pltpu.make_async_copy(ones_v, deg_sh.at[dst_v.at[j0 + b]],
                                  dsems[b]).wait()

            @pl.when(i < CH // NDB - 1)
            def _():
                pltpu.async_copy(ones_v, deg_sh.at[dst_v.at[j0 + NDB + b]],
                                 dsems[b], add=True)
        return carry

    lax.fori_loop(0, CH // NDB, body, 0)
    plsc.subcore_barrier()

    @pl.when(sid == 0)
    def _():
        pltpu.sync_copy(deg_sh, out_hbm.at[cid])


NBLK = 1000
NGRID = N // NBLK


def _mm_scale_body(x_ref, w_ref, deg_ref, o_ref, dinv_ref):
    d = deg_ref[0] + deg_ref[1] + 1.0
    dinv = lax.rsqrt(d)
    dinv_ref[...] = dinv
    h = jnp.dot(x_ref[...], w_ref[...], preferred_element_type=jnp.float32)
    o_ref[...] = h * dinv


def _tc_mm_scale(x, W, deg2):
    return pl.pallas_call(
        _mm_scale_body,
        grid=(NGRID,),
        in_specs=[
            pl.BlockSpec((NBLK, D), lambda i: (i, 0)),
            pl.BlockSpec((D, D), lambda i: (0, 0)),
            pl.BlockSpec((2, NBLK, 1), lambda i: (0, i, 0)),
        ],
        out_specs=[
            pl.BlockSpec((NBLK, D), lambda i: (i, 0)),
            pl.BlockSpec((NBLK, 1), lambda i: (i, 0)),
        ],
        out_shape=[
            jax.ShapeDtypeStruct((NP, D), jnp.float32),
            jax.ShapeDtypeStruct((NP, 1), jnp.float32),
        ],
    )(x, W, deg2)


def _combine_mm_body(acc_ref, hp_ref, dinv_ref, b_ref, w_ref, o_ref):
    dinv = dinv_ref[...]
    enc = dinv * (acc_ref[0] + acc_ref[1] + hp_ref[...]) + b_ref[...]
    o_ref[...] = jnp.dot(enc, w_ref[...],
                         preferred_element_type=jnp.float32) * dinv


def _tc_combine_mm(acc2, hp, dinv, b, W):
    return pl.pallas_call(
        _combine_mm_body,
        grid=(NGRID,),
        in_specs=[
            pl.BlockSpec((2, NBLK, D), lambda i: (0, i, 0)),
            pl.BlockSpec((NBLK, D), lambda i: (i, 0)),
            pl.BlockSpec((NBLK, 1), lambda i: (i, 0)),
            pl.BlockSpec((1, D), lambda i: (0, 0)),
            pl.BlockSpec((D, D), lambda i: (0, 0)),
        ],
        out_specs=pl.BlockSpec((NBLK, D), lambda i: (i, 0)),
        out_shape=jax.ShapeDtypeStruct((NP, D), jnp.float32),
    )(acc2, hp, dinv, b, W)


def _final_body(acc_ref, hp_ref, dinv_ref, b_ref, o_ref):
    o_ref[...] = (dinv_ref[...] * (acc_ref[0] + acc_ref[1] + hp_ref[...])
                  + b_ref[...])


def _tc_final(acc2, hp, dinv, b):
    return pl.pallas_call(
        _final_body,
        grid=(NGRID,),
        in_specs=[
            pl.BlockSpec((2, NBLK, D), lambda i: (0, i, 0)),
            pl.BlockSpec((NBLK, D), lambda i: (i, 0)),
            pl.BlockSpec((NBLK, 1), lambda i: (i, 0)),
            pl.BlockSpec((1, D), lambda i: (0, 0)),
        ],
        out_specs=pl.BlockSpec((NBLK, D), lambda i: (i, 0)),
        out_shape=jax.ShapeDtypeStruct((N, D), jnp.float32),
    )(acc2, hp, dinv, b)


def kernel(x, edge_index, W_enc, b_enc, W_dec, b_dec):
    src_p = edge_index[0].astype(jnp.int32).reshape(NW * CH, CHUNK)
    dst_p = edge_index[1].astype(jnp.int32).reshape(NW * CH, CHUNK)
    zeros = jnp.zeros((STRIPE, D), jnp.float32)
    zeros1 = jnp.zeros((STRIPE,), jnp.float32)
    ones1 = jnp.ones((CHUNK,), jnp.float32)

    deg2 = _sc_degree(dst_p, ones1, zeros1)
    h1, dinv = _tc_mm_scale(x, W_enc,
                            deg2.reshape(2, NP, 1))
    acc1 = _sc_segsum(h1, src_p, dst_p, zeros)
    h2 = _tc_combine_mm(acc1, h1, dinv,
                        b_enc.reshape(1, D), W_dec)
    acc2 = _sc_segsum(h2, src_p, dst_p, zeros)
    return _tc_final(acc2, h2, dinv, b_dec.reshape(1, D))
